# baseline (speedup 1.0000x reference)
import jax
import jax.numpy as jnp
from jax import lax
from jax.experimental import pallas as pl
from jax.experimental.pallas import tpu as pltpu
from jax.experimental.pallas import tpu_sc as plsc

N = 10000
NP = 10240
D = 128
E = 320000
NC, NS, L = 2, 16, 16
HALF = NP // NC
AR = 160
REAL_ROWS = E // 128
EPR = NS * AR
WROW = 144
CMAX = AR * 128 + 256
P = 65536
PPW = P // (NC * NS)

_mesh = plsc.VectorSubcoreMesh(
    core_axis_name="c", subcore_axis_name="s", num_cores=NC, num_subcores=NS)
_sc_params = pltpu.CompilerParams(
    needs_layout_passes=False, use_tc_tiling_on_sc=False)


def _dense_body(h_ref, w_ref, a_ref, z_ref, stm_ref):
    h = h_ref[...]
    w = w_ref[...]
    z = lax.dot_general(h, w, (((1,), (1,)), ((), ())),
                        preferred_element_type=jnp.float32)
    z_ref[...] = z
    a = a_ref[...]
    s = z @ a[0, :D]
    t = z @ a[0, D:]
    stm_ref[pl.ds(0, NP)] = s
    stm_ref[pl.ds(NP, NP)] = t
    stm_ref[pl.ds(2 * NP, L)] = jnp.broadcast_to(jnp.max(s), (L,))


def _dense_layer(h, w1, a):
    return pl.pallas_call(
        _dense_body,
        out_shape=[
            jax.ShapeDtypeStruct((NP, D), jnp.float32),
            jax.ShapeDtypeStruct((2 * NP + L,), jnp.float32),
        ],
    )(h, w1, a)


def _combine_body(part_ref, h_ref):
    acc = part_ref[...]
    num = acc[:, :D]
    den = acc[:, D:D + 1]
    h = jnp.where(den > 0, num / den, 0.0)
    h_ref[...] = jnp.maximum(h, 0.0)


def _combine(part):
    return pl.pallas_call(
        _combine_body,
        out_shape=jax.ShapeDtypeStruct((NP, D), jnp.float32),
    )(part)


def _classify_body(h_ref, wstk_ref, bvec_ref, v_ref):
    h = h_ref[...]
    v = lax.dot_general(h, wstk_ref[...], (((1,), (0,)), ((), ())),
                        preferred_element_type=jnp.float32)
    v_ref[...] = v + bvec_ref[...][None, :]


def _classify(h, wstk, bvec):
    return pl.pallas_call(
        _classify_body,
        out_shape=jax.ShapeDtypeStruct((NP, 4), jnp.float32),
    )(h, wstk, bvec)


def _edge_body(stm_hbm, src_hbm, dst_hbm, z_hbm, part_hbm,
               stm_tab, sstage, dstage, cpak, csrc2, cdst2,
               rows_g, rows_s, wbuf, acc, sem):
    c = lax.axis_index("c")
    sid = lax.axis_index("s")
    lo = c * HALF

    pltpu.sync_copy(stm_hbm, stm_tab)
    smax = stm_tab[pl.ds(2 * NP, L)][0]

    def _zero(r, carry):
        for q in range(WROW // L):
            rows_s[r, pl.ds(q * L, L)] = jnp.zeros((L,), jnp.float32)
        return carry
    lax.fori_loop(0, 128, _zero, 0)
    tile_rows = HALF // NS
    pltpu.sync_copy(rows_s, acc.at[pl.ds(sid * tile_rows, 128)])
    pltpu.sync_copy(rows_s, acc.at[pl.ds(sid * tile_rows + 128, 128)])
    pltpu.sync_copy(rows_s.at[pl.ds(0, 64)],
                    acc.at[pl.ds(sid * tile_rows + 256, 64)])
    plsc.subcore_barrier()

    ramp = lax.iota(jnp.int32, L)
    onehot = jnp.where(ramp == 0, 1.0, 0.0)

    zi = jnp.zeros((L,), jnp.int32)
    def _zc(b, carry):
        for q in range(8):
            cpak[pl.ds(b * 128 + q * L, L)] = zi
        return carry
    lax.fori_loop(0, CMAX // 128, _zc, 0)

    def _blk(blk, cnt):
        pltpu.sync_copy(src_hbm.at[pl.ds(sid * AR + blk * 16, 16)], sstage)
        pltpu.sync_copy(dst_hbm.at[pl.ds(sid * AR + blk * 16, 16)], dstage)

        def _row(rr, cnt):
            grow = sid * AR + blk * 16 + rr
            growv = jnp.broadcast_to(grow, (L,))
            for g in range(128 // L):
                s16 = sstage[rr, pl.ds(g * L, L)]
                d16 = dstage[rr, pl.ds(g * L, L)]
                keep = ((d16 >= lo) & (d16 < lo + HALF)
                        & (growv < REAL_ROWS))
                pk = s16 | ((d16 - lo) << 14)
                plsc.store_compressed(cpak.at[pl.ds(cnt, L)], pk, mask=keep)
                cnt = cnt + plsc.all_reduce_population_count(keep)[0]
            return cnt
        return lax.fori_loop(0, 16, _row, cnt)

    cnt = lax.fori_loop(0, AR // 16, _blk, jnp.int32(0))

    def _chunk(j, carry):
        for g in range(128 // L):
            off = j * 128 + g * L
            pk = cpak[pl.ds(off, L)]
            s16 = pk & 16383
            d16 = lax.shift_right_logical(pk, 14)
            csrc2[0, pl.ds(g * L, L)] = s16
            cdst2[0, pl.ds(g * L, L)] = d16
            sv = plsc.load_gather(stm_tab, [s16])
            tv = plsc.load_gather(stm_tab, [d16 + (lo + NP)])
            e = sv + tv
            e = jnp.where(e >= 0, e, 0.01 * e)
            big = smax + tv
            mv = jnp.where(big >= 0, big, 0.01 * big)
            validf = jnp.where(off + ramp < cnt, 1.0, 0.0)
            wbuf[pl.ds(g * L, L)] = jnp.exp(e - mv) * validf
        pltpu.async_copy(z_hbm.at[csrc2.at[0]], rows_g, sem).wait()

        def _scale(gg, inner):
            w16 = wbuf[pl.ds(gg * L, L)]
            for i in range(L):
                r = gg * L + i
                wr = w16[i]
                for q in range(D // L):
                    rows_s[r, pl.ds(q * L, L)] = (
                        rows_g[r, pl.ds(q * L, L)] * wr)
                rows_s[r, pl.ds(D, L)] = wr * onehot
            return inner
        lax.fori_loop(0, 128 // L, _scale, 0)
        pltpu.sync_copy(rows_s, acc.at[cdst2.at[0]], add=True)
        return carry

    lax.fori_loop(0, (cnt + 127) // 128, _chunk, 0)
    plsc.subcore_barrier()
    pltpu.sync_copy(acc.at[pl.ds(sid * tile_rows, tile_rows)],
                    part_hbm.at[pl.ds(c * HALF + sid * tile_rows, tile_rows)])


_edge_call = pl.kernel(
    _edge_body,
    out_type=jax.ShapeDtypeStruct((NP, WROW), jnp.float32),
    mesh=_mesh,
    compiler_params=_sc_params,
    scratch_types=[
        pltpu.VMEM((2 * NP + L,), jnp.float32),
        pltpu.VMEM((16, 128), jnp.int32),
        pltpu.VMEM((16, 128), jnp.int32),
        pltpu.VMEM((CMAX,), jnp.int32),
        pltpu.VMEM((1, 128), jnp.int32),
        pltpu.VMEM((1, 128), jnp.int32),
        pltpu.VMEM((128, D), jnp.float32),
        pltpu.VMEM((128, WROW), jnp.float32),
        pltpu.VMEM((128,), jnp.float32),
        pltpu.VMEM_SHARED((HALF, WROW), jnp.float32),
        pltpu.SemaphoreType.DMA,
    ],
)


def _pair_body(v_hbm, pair_hbm, out_hbm, v_tab, p0_buf, p1_buf, obuf):
    c = lax.axis_index("c")
    sid = lax.axis_index("s")
    wid = sid * NC + c
    base = wid * PPW

    pltpu.sync_copy(v_hbm, v_tab)
    pltpu.sync_copy(pair_hbm.at[0, pl.ds(base, PPW)], p0_buf)
    pltpu.sync_copy(pair_hbm.at[1, pl.ds(base, PPW)], p1_buf)

    ramp = lax.iota(jnp.int32, L)

    def _grp(k, carry):
        i0 = p0_buf[pl.ds(k * L, L)] * 4
        i1 = p1_buf[pl.ds(k * L, L)] * 4
        l0 = (plsc.load_gather(v_tab, [i0])
              + plsc.load_gather(v_tab, [i1 + 2]))
        l1 = (plsc.load_gather(v_tab, [i0 + 1])
              + plsc.load_gather(v_tab, [i1 + 3]))
        s0 = 1.0 / (1.0 + jnp.exp(-l0))
        s1 = 1.0 / (1.0 + jnp.exp(-l1))
        idx2 = (k * L + ramp) * 2
        plsc.store_scatter(obuf, [idx2], s0)
        plsc.store_scatter(obuf, [idx2 + 1], s1)
        return carry

    lax.fori_loop(0, PPW // L, _grp, 0)
    pltpu.sync_copy(obuf, out_hbm.at[pl.ds(base * 2, PPW * 2)])


_pair_call = pl.kernel(
    _pair_body,
    out_type=jax.ShapeDtypeStruct((2 * P,), jnp.float32),
    mesh=_mesh,
    compiler_params=_sc_params,
    scratch_types=[
        pltpu.VMEM((4 * NP,), jnp.float32),
        pltpu.VMEM((PPW,), jnp.int32),
        pltpu.VMEM((PPW,), jnp.int32),
        pltpu.VMEM((2 * PPW,), jnp.float32),
    ],
)


@jax.jit
def kernel(x, edge_index, pair_index, W1_0, A_0, W1_1, A_1, Wc, bc):
    xp = jnp.pad(x, ((0, NP - N), (0, 0)))
    pad_e = EPR * 128 - E
    src = jnp.pad(edge_index[0], (0, pad_e)).reshape(EPR, 128)
    dst = jnp.pad(edge_index[1], (0, pad_e)).reshape(EPR, 128)

    z1, stm1 = _dense_layer(xp, W1_0, A_0)
    part1 = _edge_call(stm1, src, dst, z1)
    h1 = _combine(part1)

    z2, stm2 = _dense_layer(h1, W1_1, A_1)
    part2 = _edge_call(stm2, src, dst, z2)
    h2 = _combine(part2)

    wstk = jnp.stack([Wc[0, :D], Wc[1, :D], Wc[0, D:], Wc[1, D:]], axis=1)
    bvec = jnp.concatenate([bc, jnp.zeros((2,), jnp.float32)])
    v = _classify(h2, wstk, bvec)

    probs_flat = _pair_call(v.reshape(-1), pair_index)
    return h2[:N], probs_flat.reshape(P, 2)


# speedup vs baseline: 1.1507x; 1.1507x over previous
import jax
import jax.numpy as jnp
from jax import lax
from jax.experimental import pallas as pl
from jax.experimental.pallas import tpu as pltpu
from jax.experimental.pallas import tpu_sc as plsc

N = 10000
NP = 10240
D = 128
E = 320000
NC, NS, L = 2, 16, 16
HALF = NP // NC
AR = 160
REAL_ROWS = E // 128
EPR = NS * AR
WROW = 144
CMAX = AR * 128 + 256
P = 65536
PPW = P // (NC * NS)

_mesh = plsc.VectorSubcoreMesh(
    core_axis_name="c", subcore_axis_name="s", num_cores=NC, num_subcores=NS)
_sc_params = pltpu.CompilerParams(
    needs_layout_passes=False, use_tc_tiling_on_sc=False)


def _dense_body(h_ref, w_ref, a_ref, z_ref, stm_ref):
    h = h_ref[...]
    w = w_ref[...]
    z = lax.dot_general(h, w, (((1,), (1,)), ((), ())),
                        preferred_element_type=jnp.float32)
    z_ref[...] = z
    a = a_ref[...]
    s = z @ a[0, :D]
    t = z @ a[0, D:]
    stm_ref[pl.ds(0, NP)] = s
    stm_ref[pl.ds(NP, NP)] = t
    stm_ref[pl.ds(2 * NP, L)] = jnp.broadcast_to(jnp.max(s), (L,))


def _dense_layer(h, w1, a):
    return pl.pallas_call(
        _dense_body,
        out_shape=[
            jax.ShapeDtypeStruct((NP, D), jnp.float32),
            jax.ShapeDtypeStruct((2 * NP + L,), jnp.float32),
        ],
    )(h, w1, a)


def _combine_body(part_ref, h_ref):
    acc = part_ref[...]
    num = acc[:, :D]
    den = acc[:, D:D + 1]
    h = jnp.where(den > 0, num / den, 0.0)
    h_ref[...] = jnp.maximum(h, 0.0)


def _combine(part):
    return pl.pallas_call(
        _combine_body,
        out_shape=jax.ShapeDtypeStruct((NP, D), jnp.float32),
    )(part)


def _classify_body(h_ref, wstk_ref, bvec_ref, v_ref):
    h = h_ref[...]
    v = lax.dot_general(h, wstk_ref[...], (((1,), (0,)), ((), ())),
                        preferred_element_type=jnp.float32)
    v_ref[...] = v + bvec_ref[...][None, :]


def _classify(h, wstk, bvec):
    return pl.pallas_call(
        _classify_body,
        out_shape=jax.ShapeDtypeStruct((NP, 4), jnp.float32),
    )(h, wstk, bvec)


def _edge_body(stm_hbm, src_hbm, dst_hbm, z_hbm, part_hbm,
               st_tab, sstage, dstage, cpak, csrc2, cdst2,
               rows_g, rows_s, wbuf, acc, sem0, sem1):
    c = lax.axis_index("c")
    sid = lax.axis_index("s")
    lo = c * HALF

    pltpu.sync_copy(stm_hbm.at[pl.ds(0, NP)], st_tab.at[pl.ds(0, NP)])
    pltpu.sync_copy(stm_hbm.at[pl.ds(NP + lo, HALF)], st_tab.at[pl.ds(NP, HALF)])
    pltpu.sync_copy(stm_hbm.at[pl.ds(2 * NP, L)], st_tab.at[pl.ds(NP + HALF, L)])
    smax = st_tab[pl.ds(NP + HALF, L)][0]

    def _zero(r, carry):
        for q in range(WROW // L):
            rows_s[r, pl.ds(q * L, L)] = jnp.zeros((L,), jnp.float32)
        return carry
    lax.fori_loop(0, 128, _zero, 0)
    tile_rows = HALF // NS
    pltpu.sync_copy(rows_s, acc.at[pl.ds(sid * tile_rows, 128)])
    pltpu.sync_copy(rows_s, acc.at[pl.ds(sid * tile_rows + 128, 128)])
    pltpu.sync_copy(rows_s.at[pl.ds(0, 64)],
                    acc.at[pl.ds(sid * tile_rows + 256, 64)])
    plsc.subcore_barrier()

    ramp = lax.iota(jnp.int32, L)
    onehot = jnp.where(ramp == 0, 1.0, 0.0)

    zi = jnp.zeros((L,), jnp.int32)
    def _zc(b, carry):
        for q in range(8):
            cpak[pl.ds(b * 128 + q * L, L)] = zi
        return carry
    lax.fori_loop(0, CMAX // 128, _zc, 0)

    def _blk(blk, cnt):
        pltpu.sync_copy(src_hbm.at[pl.ds(sid * AR + blk * 16, 16)], sstage)
        pltpu.sync_copy(dst_hbm.at[pl.ds(sid * AR + blk * 16, 16)], dstage)

        def _row(rr, cnt):
            grow = sid * AR + blk * 16 + rr
            growv = jnp.broadcast_to(grow, (L,))
            for g in range(128 // L):
                s16 = sstage[rr, pl.ds(g * L, L)]
                d16 = dstage[rr, pl.ds(g * L, L)]
                keep = ((d16 >= lo) & (d16 < lo + HALF)
                        & (growv < REAL_ROWS))
                pk = s16 | ((d16 - lo) << 14)
                plsc.store_compressed(cpak.at[pl.ds(cnt, L)], pk, mask=keep)
                cnt = cnt + plsc.all_reduce_population_count(keep)[0]
            return cnt
        return lax.fori_loop(0, 16, _row, cnt)

    cnt = lax.fori_loop(0, AR // 16, _blk, jnp.int32(0))

    def _prep(jj, par, pp, psem):
        for g in range(64 // L):
            off = jj * 64 + g * L
            pk = cpak[pl.ds(off, L)]
            s16 = pk & 16383
            d16 = lax.shift_right_logical(pk, 14)
            csrc2[par, pl.ds(g * L, L)] = s16
            cdst2[pp, pl.ds(par * 64 + g * L, L)] = d16
            sv = plsc.load_gather(st_tab, [s16])
            tv = plsc.load_gather(st_tab, [d16 + NP])
            e = sv + tv
            e = jnp.where(e >= 0, e, 0.01 * e)
            big = smax + tv
            mv = jnp.where(big >= 0, big, 0.01 * big)
            validf = jnp.where(off + ramp < cnt, 1.0, 0.0)
            wbuf[par, pl.ds(g * L, L)] = jnp.exp(e - mv) * validf
        pltpu.async_copy(z_hbm.at[csrc2.at[par]], rows_g.at[par], psem)

    def _wait(par, psem):
        pltpu.make_async_copy(z_hbm.at[csrc2.at[par]], rows_g.at[par],
                              psem).wait()

    def _scale(par):
        def body(gg, inner):
            w16 = wbuf[par, pl.ds(gg * L, L)]
            for i in range(L):
                rloc = gg * L + i
                wr = w16[i]
                for q in range(D // L):
                    rows_s[par * 64 + rloc, pl.ds(q * L, L)] = (
                        rows_g[par, rloc, pl.ds(q * L, L)] * wr)
                rows_s[par * 64 + rloc, pl.ds(D, L)] = wr * onehot
            return inner
        lax.fori_loop(0, 64 // L, body, 0)

    npairs = (cnt + 127) // 128

    @pl.when(npairs > 0)
    def _prologue():
        _prep(0, 0, jnp.int32(0), sem0)

    def _pair(p, carry):
        pp = p & 1
        _prep(2 * p + 1, 1, pp, sem1)
        _wait(0, sem0)
        _scale(0)

        @pl.when(p + 1 < npairs)
        def _prefetch():
            _prep(2 * p + 2, 0, 1 - pp, sem0)

        _wait(1, sem1)
        _scale(1)
        pltpu.sync_copy(rows_s, acc.at[cdst2.at[pp]], add=True)
        return carry

    lax.fori_loop(0, npairs, _pair, 0)
    plsc.subcore_barrier()
    pltpu.sync_copy(acc.at[pl.ds(sid * tile_rows, tile_rows)],
                    part_hbm.at[pl.ds(c * HALF + sid * tile_rows, tile_rows)])


_edge_call = pl.kernel(
    _edge_body,
    out_type=jax.ShapeDtypeStruct((NP, WROW), jnp.float32),
    mesh=_mesh,
    compiler_params=_sc_params,
    scratch_types=[
        pltpu.VMEM((NP + HALF + L,), jnp.float32),
        pltpu.VMEM((16, 128), jnp.int32),
        pltpu.VMEM((16, 128), jnp.int32),
        pltpu.VMEM((CMAX,), jnp.int32),
        pltpu.VMEM((2, 64), jnp.int32),
        pltpu.VMEM((2, 128), jnp.int32),
        pltpu.VMEM((2, 64, D), jnp.float32),
        pltpu.VMEM((128, WROW), jnp.float32),
        pltpu.VMEM((2, 64), jnp.float32),
        pltpu.VMEM_SHARED((HALF, WROW), jnp.float32),
        pltpu.SemaphoreType.DMA,
        pltpu.SemaphoreType.DMA,
    ],
)


def _pair_body(v_hbm, pair_hbm, out_hbm, v_tab, p0_buf, p1_buf, obuf):
    c = lax.axis_index("c")
    sid = lax.axis_index("s")
    wid = sid * NC + c
    base = wid * PPW

    pltpu.sync_copy(v_hbm, v_tab)
    pltpu.sync_copy(pair_hbm.at[0, pl.ds(base, PPW)], p0_buf)
    pltpu.sync_copy(pair_hbm.at[1, pl.ds(base, PPW)], p1_buf)

    ramp = lax.iota(jnp.int32, L)

    def _grp(k, carry):
        i0 = p0_buf[pl.ds(k * L, L)] * 4
        i1 = p1_buf[pl.ds(k * L, L)] * 4
        l0 = (plsc.load_gather(v_tab, [i0])
              + plsc.load_gather(v_tab, [i1 + 2]))
        l1 = (plsc.load_gather(v_tab, [i0 + 1])
              + plsc.load_gather(v_tab, [i1 + 3]))
        s0 = 1.0 / (1.0 + jnp.exp(-l0))
        s1 = 1.0 / (1.0 + jnp.exp(-l1))
        idx2 = (k * L + ramp) * 2
        plsc.store_scatter(obuf, [idx2], s0)
        plsc.store_scatter(obuf, [idx2 + 1], s1)
        return carry

    lax.fori_loop(0, PPW // L, _grp, 0)
    pltpu.sync_copy(obuf, out_hbm.at[pl.ds(base * 2, PPW * 2)])


_pair_call = pl.kernel(
    _pair_body,
    out_type=jax.ShapeDtypeStruct((2 * P,), jnp.float32),
    mesh=_mesh,
    compiler_params=_sc_params,
    scratch_types=[
        pltpu.VMEM((4 * NP,), jnp.float32),
        pltpu.VMEM((PPW,), jnp.int32),
        pltpu.VMEM((PPW,), jnp.int32),
        pltpu.VMEM((2 * PPW,), jnp.float32),
    ],
)


@jax.jit
def kernel(x, edge_index, pair_index, W1_0, A_0, W1_1, A_1, Wc, bc):
    xp = jnp.pad(x, ((0, NP - N), (0, 0)))
    pad_e = EPR * 128 - E
    src = jnp.pad(edge_index[0], (0, pad_e)).reshape(EPR, 128)
    dst = jnp.pad(edge_index[1], (0, pad_e)).reshape(EPR, 128)

    z1, stm1 = _dense_layer(xp, W1_0, A_0)
    part1 = _edge_call(stm1, src, dst, z1)
    h1 = _combine(part1)

    z2, stm2 = _dense_layer(h1, W1_1, A_1)
    part2 = _edge_call(stm2, src, dst, z2)
    h2 = _combine(part2)

    wstk = jnp.stack([Wc[0, :D], Wc[1, :D], Wc[0, D:], Wc[1, D:]], axis=1)
    bvec = jnp.concatenate([bc, jnp.zeros((2,), jnp.float32)])
    v = _classify(h2, wstk, bvec)

    probs_flat = _pair_call(v.reshape(-1), pair_index)
    return h2[:N], probs_flat.reshape(P, 2)


# speedup vs baseline: 1.1648x; 1.0122x over previous
import jax
import jax.numpy as jnp
from jax import lax
from jax.experimental import pallas as pl
from jax.experimental.pallas import tpu as pltpu
from jax.experimental.pallas import tpu_sc as plsc

N = 10000
NP = 10240
D = 128
E = 320000
NC, NS, L = 2, 16, 16
HALF = NP // NC
AR = 160
REAL_ROWS = E // 128
EPR = NS * AR
WROW = 144
CMAX = AR * 128 + 256
P = 65536
PPW = P // (NC * NS)

_mesh = plsc.VectorSubcoreMesh(
    core_axis_name="c", subcore_axis_name="s", num_cores=NC, num_subcores=NS)
_sc_params = pltpu.CompilerParams(
    needs_layout_passes=False, use_tc_tiling_on_sc=False)


def _dense_body(h_ref, w_ref, a_ref, z_ref, stm_ref):
    h = h_ref[...]
    w = w_ref[...]
    z = lax.dot_general(h, w, (((1,), (1,)), ((), ())),
                        preferred_element_type=jnp.float32)
    z_ref[...] = z
    a = a_ref[...]
    s = z @ a[0, :D]
    t = z @ a[0, D:]
    stm_ref[pl.ds(0, NP)] = s
    stm_ref[pl.ds(NP, NP)] = t
    stm_ref[pl.ds(2 * NP, L)] = jnp.broadcast_to(jnp.max(s), (L,))


def _dense_layer(h, w1, a):
    return pl.pallas_call(
        _dense_body,
        out_shape=[
            jax.ShapeDtypeStruct((NP, D), jnp.float32),
            jax.ShapeDtypeStruct((2 * NP + L,), jnp.float32),
        ],
    )(h, w1, a)


def _combine_body(part_ref, h_ref):
    acc = part_ref[...]
    num = acc[:, :D]
    den = acc[:, D:D + 1]
    h = jnp.where(den > 0, num / den, 0.0)
    h_ref[...] = jnp.maximum(h, 0.0)


def _combine(part):
    return pl.pallas_call(
        _combine_body,
        out_shape=jax.ShapeDtypeStruct((NP, D), jnp.float32),
    )(part)


def _classify_body(h_ref, wstk_ref, bvec_ref, v_ref):
    h = h_ref[...]
    v = lax.dot_general(h, wstk_ref[...], (((1,), (0,)), ((), ())),
                        preferred_element_type=jnp.float32)
    v_ref[...] = v + bvec_ref[...][None, :]


def _classify(h, wstk, bvec):
    return pl.pallas_call(
        _classify_body,
        out_shape=jax.ShapeDtypeStruct((NP, 4), jnp.float32),
    )(h, wstk, bvec)


def _edge_body(stm_hbm, src_hbm, dst_hbm, z_hbm, part_hbm,
               st_tab, sstage, dstage, cpak, csrc2, cdst2,
               rows_g, rows_s, wbuf, acc, sem0, sem1, sem2):
    c = lax.axis_index("c")
    sid = lax.axis_index("s")
    lo = c * HALF

    pltpu.sync_copy(stm_hbm.at[pl.ds(0, NP)], st_tab.at[pl.ds(0, NP)])
    pltpu.sync_copy(stm_hbm.at[pl.ds(NP + lo, HALF)], st_tab.at[pl.ds(NP, HALF)])
    pltpu.sync_copy(stm_hbm.at[pl.ds(2 * NP, L)], st_tab.at[pl.ds(NP + HALF, L)])
    smax = st_tab[pl.ds(NP + HALF, L)][0]

    def _zero(r, carry):
        for q in range(WROW // L):
            rows_s[r, pl.ds(q * L, L)] = jnp.zeros((L,), jnp.float32)
        return carry
    lax.fori_loop(0, 128, _zero, 0)
    tile_rows = HALF // NS
    pltpu.sync_copy(rows_s, acc.at[pl.ds(sid * tile_rows, 128)])
    pltpu.sync_copy(rows_s, acc.at[pl.ds(sid * tile_rows + 128, 128)])
    pltpu.sync_copy(rows_s.at[pl.ds(0, 64)],
                    acc.at[pl.ds(sid * tile_rows + 256, 64)])
    plsc.subcore_barrier()

    ramp = lax.iota(jnp.int32, L)
    onehot = jnp.where(ramp == 0, 1.0, 0.0)

    zi = jnp.zeros((L,), jnp.int32)
    def _zc(b, carry):
        for q in range(8):
            cpak[pl.ds(b * 128 + q * L, L)] = zi
        return carry
    lax.fori_loop(0, CMAX // 128, _zc, 0)

    def _blk(blk, cnt):
        pltpu.sync_copy(src_hbm.at[pl.ds(sid * AR + blk * 16, 16)], sstage)
        pltpu.sync_copy(dst_hbm.at[pl.ds(sid * AR + blk * 16, 16)], dstage)

        def _row(rr, cnt):
            grow = sid * AR + blk * 16 + rr
            growv = jnp.broadcast_to(grow, (L,))
            for g in range(128 // L):
                s16 = sstage[rr, pl.ds(g * L, L)]
                d16 = dstage[rr, pl.ds(g * L, L)]
                keep = ((d16 >= lo) & (d16 < lo + HALF)
                        & (growv < REAL_ROWS))
                pk = s16 | ((d16 - lo) << 14)
                plsc.store_compressed(cpak.at[pl.ds(cnt, L)], pk, mask=keep)
                cnt = cnt + plsc.all_reduce_population_count(keep)[0]
            return cnt
        return lax.fori_loop(0, 16, _row, cnt)

    cnt = lax.fori_loop(0, AR // 16, _blk, jnp.int32(0))

    def _prep(jj, par, pp, psem):
        for g in range(64 // L):
            off = jj * 64 + g * L
            pk = cpak[pl.ds(off, L)]
            s16 = pk & 16383
            d16 = lax.shift_right_logical(pk, 14)
            csrc2[par, pl.ds(g * L, L)] = s16
            cdst2[pp, pl.ds(par * 64 + g * L, L)] = d16
            sv = plsc.load_gather(st_tab, [s16])
            tv = plsc.load_gather(st_tab, [d16 + NP])
            e = sv + tv
            e = jnp.where(e >= 0, e, 0.01 * e)
            big = smax + tv
            mv = jnp.where(big >= 0, big, 0.01 * big)
            validf = jnp.where(off + ramp < cnt, 1.0, 0.0)
            wbuf[par, pl.ds(g * L, L)] = jnp.exp(e - mv) * validf
        pltpu.async_copy(z_hbm.at[csrc2.at[par]], rows_g.at[par], psem)

    def _wait(par, psem):
        pltpu.make_async_copy(z_hbm.at[csrc2.at[par]], rows_g.at[par],
                              psem).wait()

    def _scale(par):
        def body(gg, inner):
            w16 = wbuf[par, pl.ds(gg * L, L)]
            for i in range(L):
                rloc = gg * L + i
                wr = w16[i]
                for q in range(D // L):
                    rows_s[par * 64 + rloc, pl.ds(q * L, L)] = (
                        rows_g[par, rloc, pl.ds(q * L, L)] * wr)
                rows_s[par * 64 + rloc, pl.ds(D, L)] = wr * onehot
            return inner
        lax.fori_loop(0, 64 // L, body, 0)

    npairs = (cnt + 127) // 128

    def _wait_scatter(ppx):
        pltpu.make_async_copy(rows_s, acc.at[cdst2.at[ppx]], sem2).wait()

    @pl.when(npairs > 0)
    def _prologue():
        _prep(0, 0, jnp.int32(0), sem0)

    def _pair(p, carry):
        pp = p & 1
        _prep(2 * p + 1, 1, pp, sem1)
        _wait(0, sem0)

        @pl.when(p > 0)
        def _drain_prev():
            _wait_scatter(1 - pp)

        _scale(0)

        @pl.when(p + 1 < npairs)
        def _prefetch():
            _prep(2 * p + 2, 0, 1 - pp, sem0)

        _wait(1, sem1)
        _scale(1)
        pltpu.async_copy(rows_s, acc.at[cdst2.at[pp]], sem2, add=True)
        return carry

    lax.fori_loop(0, npairs, _pair, 0)

    @pl.when(npairs > 0)
    def _drain_last():
        _wait_scatter((npairs - 1) & 1)
    plsc.subcore_barrier()
    pltpu.sync_copy(acc.at[pl.ds(sid * tile_rows, tile_rows)],
                    part_hbm.at[pl.ds(c * HALF + sid * tile_rows, tile_rows)])


_edge_call = pl.kernel(
    _edge_body,
    out_type=jax.ShapeDtypeStruct((NP, WROW), jnp.float32),
    mesh=_mesh,
    compiler_params=_sc_params,
    scratch_types=[
        pltpu.VMEM((NP + HALF + L,), jnp.float32),
        pltpu.VMEM((16, 128), jnp.int32),
        pltpu.VMEM((16, 128), jnp.int32),
        pltpu.VMEM((CMAX,), jnp.int32),
        pltpu.VMEM((2, 64), jnp.int32),
        pltpu.VMEM((2, 128), jnp.int32),
        pltpu.VMEM((2, 64, D), jnp.float32),
        pltpu.VMEM((128, WROW), jnp.float32),
        pltpu.VMEM((2, 64), jnp.float32),
        pltpu.VMEM_SHARED((HALF, WROW), jnp.float32),
        pltpu.SemaphoreType.DMA,
        pltpu.SemaphoreType.DMA,
        pltpu.SemaphoreType.DMA,
    ],
)


def _pair_body(v_hbm, pair_hbm, out_hbm, v_tab, p0_buf, p1_buf, obuf):
    c = lax.axis_index("c")
    sid = lax.axis_index("s")
    wid = sid * NC + c
    base = wid * PPW

    pltpu.sync_copy(v_hbm, v_tab)
    pltpu.sync_copy(pair_hbm.at[0, pl.ds(base, PPW)], p0_buf)
    pltpu.sync_copy(pair_hbm.at[1, pl.ds(base, PPW)], p1_buf)

    ramp = lax.iota(jnp.int32, L)

    def _grp(k, carry):
        i0 = p0_buf[pl.ds(k * L, L)] * 4
        i1 = p1_buf[pl.ds(k * L, L)] * 4
        l0 = (plsc.load_gather(v_tab, [i0])
              + plsc.load_gather(v_tab, [i1 + 2]))
        l1 = (plsc.load_gather(v_tab, [i0 + 1])
              + plsc.load_gather(v_tab, [i1 + 3]))
        s0 = 1.0 / (1.0 + jnp.exp(-l0))
        s1 = 1.0 / (1.0 + jnp.exp(-l1))
        idx2 = (k * L + ramp) * 2
        plsc.store_scatter(obuf, [idx2], s0)
        plsc.store_scatter(obuf, [idx2 + 1], s1)
        return carry

    lax.fori_loop(0, PPW // L, _grp, 0)
    pltpu.sync_copy(obuf, out_hbm.at[pl.ds(base * 2, PPW * 2)])


_pair_call = pl.kernel(
    _pair_body,
    out_type=jax.ShapeDtypeStruct((2 * P,), jnp.float32),
    mesh=_mesh,
    compiler_params=_sc_params,
    scratch_types=[
        pltpu.VMEM((4 * NP,), jnp.float32),
        pltpu.VMEM((PPW,), jnp.int32),
        pltpu.VMEM((PPW,), jnp.int32),
        pltpu.VMEM((2 * PPW,), jnp.float32),
    ],
)


@jax.jit
def kernel(x, edge_index, pair_index, W1_0, A_0, W1_1, A_1, Wc, bc):
    xp = jnp.pad(x, ((0, NP - N), (0, 0)))
    pad_e = EPR * 128 - E
    src = jnp.pad(edge_index[0], (0, pad_e)).reshape(EPR, 128)
    dst = jnp.pad(edge_index[1], (0, pad_e)).reshape(EPR, 128)

    z1, stm1 = _dense_layer(xp, W1_0, A_0)
    part1 = _edge_call(stm1, src, dst, z1)
    h1 = _combine(part1)

    z2, stm2 = _dense_layer(h1, W1_1, A_1)
    part2 = _edge_call(stm2, src, dst, z2)
    h2 = _combine(part2)

    wstk = jnp.stack([Wc[0, :D], Wc[1, :D], Wc[0, D:], Wc[1, D:]], axis=1)
    bvec = jnp.concatenate([bc, jnp.zeros((2,), jnp.float32)])
    v = _classify(h2, wstk, bvec)

    probs_flat = _pair_call(v.reshape(-1), pair_index)
    return h2[:N], probs_flat.reshape(P, 2)


# speedup vs baseline: 2.2272x; 1.9121x over previous
import jax
import jax.numpy as jnp
from jax import lax
from jax.experimental import pallas as pl
from jax.experimental.pallas import tpu as pltpu
from jax.experimental.pallas import tpu_sc as plsc

N = 10000
NP = 10240
D = 128
E = 320000
NC, NS, L = 2, 16, 16
HALF = NP // NC
AR = 160
REAL_ROWS = E // 128
EPR = NS * AR
WROW = 144
CMAX = AR * 128 + 256
P = 65536
PPW = P // (NC * NS)

_mesh = plsc.VectorSubcoreMesh(
    core_axis_name="c", subcore_axis_name="s", num_cores=NC, num_subcores=NS)
_sc_params = pltpu.CompilerParams(
    needs_layout_passes=False, use_tc_tiling_on_sc=False)


def _dense_body(h_ref, w_ref, a_ref, z_ref, stm_ref):
    h = h_ref[...]
    w = w_ref[...]
    z = lax.dot_general(h, w, (((1,), (1,)), ((), ())),
                        preferred_element_type=jnp.float32)
    z_ref[...] = z
    a = a_ref[...]
    s = z @ a[0, :D]
    t = z @ a[0, D:]
    stm_ref[pl.ds(0, NP)] = s
    stm_ref[pl.ds(NP, NP)] = t
    stm_ref[pl.ds(2 * NP, L)] = jnp.broadcast_to(jnp.max(s), (L,))


def _dense_layer(h, w1, a):
    return pl.pallas_call(
        _dense_body,
        out_shape=[
            jax.ShapeDtypeStruct((NP, D), jnp.float32),
            jax.ShapeDtypeStruct((2 * NP + L,), jnp.float32),
        ],
    )(h, w1, a)


def _combine_body(part_ref, h_ref):
    acc = part_ref[...]
    num = acc[:, :D]
    den = acc[:, D:D + 1]
    h = jnp.where(den > 0, num / den, 0.0)
    h_ref[...] = jnp.maximum(h, 0.0)


def _combine(part):
    return pl.pallas_call(
        _combine_body,
        out_shape=jax.ShapeDtypeStruct((NP, D), jnp.float32),
    )(part)


def _classify_body(h_ref, wstk_ref, bvec_ref, v_ref):
    h = h_ref[...]
    v = lax.dot_general(h, wstk_ref[...], (((1,), (0,)), ((), ())),
                        preferred_element_type=jnp.float32)
    v_ref[...] = v + bvec_ref[...][None, :]


def _classify(h, wstk, bvec):
    return pl.pallas_call(
        _classify_body,
        out_shape=jax.ShapeDtypeStruct((NP, 4), jnp.float32),
    )(h, wstk, bvec)


def _edge_body(stm_hbm, src_hbm, dst_hbm, z_hbm, part_hbm,
               st_tab, sstage, dstage, cpak, csrc2, cdst2,
               rows_g, rows_s, wbuf, acc, sem0, sem1, sem2):
    c = lax.axis_index("c")
    sid = lax.axis_index("s")
    lo = c * HALF

    pltpu.sync_copy(stm_hbm.at[pl.ds(0, NP)], st_tab.at[pl.ds(0, NP)])
    pltpu.sync_copy(stm_hbm.at[pl.ds(NP + lo, HALF)], st_tab.at[pl.ds(NP, HALF)])
    pltpu.sync_copy(stm_hbm.at[pl.ds(2 * NP, L)], st_tab.at[pl.ds(NP + HALF, L)])
    smax = st_tab[pl.ds(NP + HALF, L)][0]

    def _zero(r, carry):
        for q in range(WROW // L):
            rows_s[r, pl.ds(q * L, L)] = jnp.zeros((L,), jnp.float32)
        return carry
    lax.fori_loop(0, 128, _zero, 0)
    tile_rows = HALF // NS
    pltpu.sync_copy(rows_s, acc.at[pl.ds(sid * tile_rows, 128)])
    pltpu.sync_copy(rows_s, acc.at[pl.ds(sid * tile_rows + 128, 128)])
    pltpu.sync_copy(rows_s.at[pl.ds(0, 64)],
                    acc.at[pl.ds(sid * tile_rows + 256, 64)])
    plsc.subcore_barrier()

    ramp = lax.iota(jnp.int32, L)
    onehot = jnp.where(ramp == 0, 1.0, 0.0)

    zi = jnp.zeros((L,), jnp.int32)
    def _zc(b, carry):
        for q in range(8):
            cpak[pl.ds(b * 128 + q * L, L)] = zi
        return carry
    lax.fori_loop(0, CMAX // 128, _zc, 0)

    def _blk(blk, cnt):
        pltpu.sync_copy(src_hbm.at[pl.ds(sid * AR + blk * 16, 16)], sstage)
        pltpu.sync_copy(dst_hbm.at[pl.ds(sid * AR + blk * 16, 16)], dstage)

        def _row(rr, cnt):
            grow = sid * AR + blk * 16 + rr
            growv = jnp.broadcast_to(grow, (L,))
            for g in range(128 // L):
                s16 = sstage[rr, pl.ds(g * L, L)]
                d16 = dstage[rr, pl.ds(g * L, L)]
                keep = ((d16 >= lo) & (d16 < lo + HALF)
                        & (growv < REAL_ROWS))
                pk = s16 | ((d16 - lo) << 14)
                plsc.store_compressed(cpak.at[pl.ds(cnt, L)], pk, mask=keep)
                cnt = cnt + plsc.all_reduce_population_count(keep)[0]
            return cnt
        return lax.fori_loop(0, 16, _row, cnt)

    cnt = lax.fori_loop(0, AR // 16, _blk, jnp.int32(0))

    def _prep(jj, par, pp, psem):
        for g in range(64 // L):
            off = jj * 64 + g * L
            pk = cpak[pl.ds(off, L)]
            s16 = pk & 16383
            d16 = lax.shift_right_logical(pk, 14)
            csrc2[par, pl.ds(g * L, L)] = s16
            cdst2[pp, pl.ds(par * 64 + g * L, L)] = d16
            sv = plsc.load_gather(st_tab, [s16])
            tv = plsc.load_gather(st_tab, [d16 + NP])
            e = sv + tv
            e = jnp.where(e >= 0, e, 0.01 * e)
            big = smax + tv
            mv = jnp.where(big >= 0, big, 0.01 * big)
            validf = jnp.where(off + ramp < cnt, 1.0, 0.0)
            wbuf[pl.ds(par * 64 + g * L, L)] = jnp.exp(e - mv) * validf
        pltpu.async_copy(z_hbm.at[csrc2.at[par]], rows_g.at[par], psem)

    def _wait(par, psem):
        pltpu.make_async_copy(z_hbm.at[csrc2.at[par]], rows_g.at[par],
                              psem).wait()

    def _scale(par):
        for gg in range(4):
            w16 = wbuf[pl.ds(par * 64 + gg * L, L)]
            for i in range(L):
                rloc = gg * L + i
                wr = w16[i]
                for q in range(D // L):
                    rows_s[par * 64 + rloc, pl.ds(q * L, L)] = (
                        rows_g[par, rloc, pl.ds(q * L, L)] * wr)
                rows_s[par * 64 + rloc, pl.ds(D, L)] = wr * onehot

    npairs = (cnt + 127) // 128

    def _wait_scatter(ppx):
        pltpu.make_async_copy(rows_s, acc.at[cdst2.at[ppx]], sem2).wait()

    @pl.when(npairs > 0)
    def _prologue():
        _prep(0, 0, jnp.int32(0), sem0)

    def _pair(p, carry):
        pp = p & 1
        _prep(2 * p + 1, 1, pp, sem1)
        _wait(0, sem0)

        @pl.when(p > 0)
        def _drain_prev():
            _wait_scatter(1 - pp)

        _scale(0)

        @pl.when(p + 1 < npairs)
        def _prefetch():
            _prep(2 * p + 2, 0, 1 - pp, sem0)

        _wait(1, sem1)
        _scale(1)
        pltpu.async_copy(rows_s, acc.at[cdst2.at[pp]], sem2, add=True)
        return carry

    lax.fori_loop(0, npairs, _pair, 0)

    @pl.when(npairs > 0)
    def _drain_last():
        _wait_scatter((npairs - 1) & 1)
    plsc.subcore_barrier()
    pltpu.sync_copy(acc.at[pl.ds(sid * tile_rows, tile_rows)],
                    part_hbm.at[pl.ds(c * HALF + sid * tile_rows, tile_rows)])


_edge_call = pl.kernel(
    _edge_body,
    out_type=jax.ShapeDtypeStruct((NP, WROW), jnp.float32),
    mesh=_mesh,
    compiler_params=_sc_params,
    scratch_types=[
        pltpu.VMEM((NP + HALF + L,), jnp.float32),
        pltpu.VMEM((16, 128), jnp.int32),
        pltpu.VMEM((16, 128), jnp.int32),
        pltpu.VMEM((CMAX,), jnp.int32),
        pltpu.VMEM((2, 64), jnp.int32),
        pltpu.VMEM((2, 128), jnp.int32),
        pltpu.VMEM((2, 64, D), jnp.float32),
        pltpu.VMEM((128, WROW), jnp.float32),
        pltpu.VMEM((128,), jnp.float32),
        pltpu.VMEM_SHARED((HALF, WROW), jnp.float32),
        pltpu.SemaphoreType.DMA,
        pltpu.SemaphoreType.DMA,
        pltpu.SemaphoreType.DMA,
    ],
)


def _pair_body(v_hbm, pair_hbm, out_hbm, v_tab, p0_buf, p1_buf, obuf):
    c = lax.axis_index("c")
    sid = lax.axis_index("s")
    wid = sid * NC + c
    base = wid * PPW

    pltpu.sync_copy(v_hbm, v_tab)
    pltpu.sync_copy(pair_hbm.at[0, pl.ds(base, PPW)], p0_buf)
    pltpu.sync_copy(pair_hbm.at[1, pl.ds(base, PPW)], p1_buf)

    ramp = lax.iota(jnp.int32, L)

    def _grp(k, carry):
        i0 = p0_buf[pl.ds(k * L, L)] * 4
        i1 = p1_buf[pl.ds(k * L, L)] * 4
        l0 = (plsc.load_gather(v_tab, [i0])
              + plsc.load_gather(v_tab, [i1 + 2]))
        l1 = (plsc.load_gather(v_tab, [i0 + 1])
              + plsc.load_gather(v_tab, [i1 + 3]))
        s0 = 1.0 / (1.0 + jnp.exp(-l0))
        s1 = 1.0 / (1.0 + jnp.exp(-l1))
        idx2 = (k * L + ramp) * 2
        plsc.store_scatter(obuf, [idx2], s0)
        plsc.store_scatter(obuf, [idx2 + 1], s1)
        return carry

    lax.fori_loop(0, PPW // L, _grp, 0)
    pltpu.sync_copy(obuf, out_hbm.at[pl.ds(base * 2, PPW * 2)])


_pair_call = pl.kernel(
    _pair_body,
    out_type=jax.ShapeDtypeStruct((2 * P,), jnp.float32),
    mesh=_mesh,
    compiler_params=_sc_params,
    scratch_types=[
        pltpu.VMEM((4 * NP,), jnp.float32),
        pltpu.VMEM((PPW,), jnp.int32),
        pltpu.VMEM((PPW,), jnp.int32),
        pltpu.VMEM((2 * PPW,), jnp.float32),
    ],
)


@jax.jit
def kernel(x, edge_index, pair_index, W1_0, A_0, W1_1, A_1, Wc, bc):
    xp = jnp.pad(x, ((0, NP - N), (0, 0)))
    pad_e = EPR * 128 - E
    src = jnp.pad(edge_index[0], (0, pad_e)).reshape(EPR, 128)
    dst = jnp.pad(edge_index[1], (0, pad_e)).reshape(EPR, 128)

    z1, stm1 = _dense_layer(xp, W1_0, A_0)
    part1 = _edge_call(stm1, src, dst, z1)
    h1 = _combine(part1)

    z2, stm2 = _dense_layer(h1, W1_1, A_1)
    part2 = _edge_call(stm2, src, dst, z2)
    h2 = _combine(part2)

    wstk = jnp.stack([Wc[0, :D], Wc[1, :D], Wc[0, D:], Wc[1, D:]], axis=1)
    bvec = jnp.concatenate([bc, jnp.zeros((2,), jnp.float32)])
    v = _classify(h2, wstk, bvec)

    probs_flat = _pair_call(v.reshape(-1), pair_index)
    return h2[:N], probs_flat.reshape(P, 2)


# speedup vs baseline: 2.2273x; 1.0000x over previous
import jax
import jax.numpy as jnp
from jax import lax
from jax.experimental import pallas as pl
from jax.experimental.pallas import tpu as pltpu
from jax.experimental.pallas import tpu_sc as plsc

N = 10000
NP = 10240
D = 128
E = 320000
NC, NS, L = 2, 16, 16
HALF = NP // NC
AR = 160
REAL_ROWS = E // 128
EPR = NS * AR
WROW = 144
CMAX = AR * 128 + 256
P = 65536
PPW = P // (NC * NS)

_mesh = plsc.VectorSubcoreMesh(
    core_axis_name="c", subcore_axis_name="s", num_cores=NC, num_subcores=NS)
_sc_params = pltpu.CompilerParams(
    needs_layout_passes=False, use_tc_tiling_on_sc=False)


def _dense_body(h_ref, w_ref, a_ref, z_ref, stm_ref):
    h = h_ref[...]
    w = w_ref[...]
    z = lax.dot_general(h, w, (((1,), (1,)), ((), ())),
                        preferred_element_type=jnp.float32)
    z_ref[...] = z
    a = a_ref[...]
    s = z @ a[0, :D]
    t = z @ a[0, D:]
    stm_ref[pl.ds(0, NP)] = s
    stm_ref[pl.ds(NP, NP)] = t
    stm_ref[pl.ds(2 * NP, L)] = jnp.broadcast_to(jnp.max(s), (L,))


def _dense_layer(h, w1, a):
    return pl.pallas_call(
        _dense_body,
        out_shape=[
            jax.ShapeDtypeStruct((NP, D), jnp.float32),
            jax.ShapeDtypeStruct((2 * NP + L,), jnp.float32),
        ],
    )(h, w1, a)


def _combine_body(part_ref, h_ref):
    acc = part_ref[...]
    num = acc[:, :D]
    den = acc[:, D:D + 1]
    h = jnp.where(den > 0, num / den, 0.0)
    h_ref[...] = jnp.maximum(h, 0.0)


def _combine(part):
    return pl.pallas_call(
        _combine_body,
        out_shape=jax.ShapeDtypeStruct((NP, D), jnp.float32),
    )(part)


def _classify_body(h_ref, wstk_ref, bvec_ref, v_ref):
    h = h_ref[...]
    v = lax.dot_general(h, wstk_ref[...], (((1,), (0,)), ((), ())),
                        preferred_element_type=jnp.float32)
    v_ref[...] = v + bvec_ref[...][None, :]


def _classify(h, wstk, bvec):
    return pl.pallas_call(
        _classify_body,
        out_shape=jax.ShapeDtypeStruct((NP, 4), jnp.float32),
    )(h, wstk, bvec)


def _edge_body(stm_hbm, src_hbm, dst_hbm, z_hbm, part_hbm,
               st_tab, sstage, dstage, cpak, csrc2, cdst2,
               rows_g, rows_s, wbuf, acc, sem0, sem1, sem2):
    c = lax.axis_index("c")
    sid = lax.axis_index("s")
    lo = c * HALF

    pltpu.sync_copy(stm_hbm.at[pl.ds(0, NP)], st_tab.at[pl.ds(0, NP)])
    pltpu.sync_copy(stm_hbm.at[pl.ds(NP + lo, HALF)], st_tab.at[pl.ds(NP, HALF)])
    pltpu.sync_copy(stm_hbm.at[pl.ds(2 * NP, L)], st_tab.at[pl.ds(NP + HALF, L)])
    smax = st_tab[pl.ds(NP + HALF, L)][0]

    def _zero(r, carry):
        for q in range(WROW // L):
            rows_s[r, pl.ds(q * L, L)] = jnp.zeros((L,), jnp.float32)
        return carry
    lax.fori_loop(0, 128, _zero, 0)
    tile_rows = HALF // NS
    pltpu.sync_copy(rows_s, acc.at[pl.ds(sid * tile_rows, 128)])
    pltpu.sync_copy(rows_s, acc.at[pl.ds(sid * tile_rows + 128, 128)])
    pltpu.sync_copy(rows_s.at[pl.ds(0, 64)],
                    acc.at[pl.ds(sid * tile_rows + 256, 64)])
    plsc.subcore_barrier()

    ramp = lax.iota(jnp.int32, L)
    onehot = jnp.where(ramp == 0, 1.0, 0.0)

    zi = jnp.zeros((L,), jnp.int32)
    def _zc(b, carry):
        for q in range(8):
            cpak[pl.ds(b * 128 + q * L, L)] = zi
        return carry
    lax.fori_loop(0, CMAX // 128, _zc, 0)

    def _blk(blk, cnt):
        pltpu.sync_copy(src_hbm.at[pl.ds(sid * AR + blk * 16, 16)], sstage)
        pltpu.sync_copy(dst_hbm.at[pl.ds(sid * AR + blk * 16, 16)], dstage)

        for rr in range(16):
            grow = sid * AR + blk * 16 + rr
            growv = jnp.broadcast_to(grow, (L,))
            for g in range(128 // L):
                s16 = sstage[rr, pl.ds(g * L, L)]
                d16 = dstage[rr, pl.ds(g * L, L)]
                keep = ((d16 >= lo) & (d16 < lo + HALF)
                        & (growv < REAL_ROWS))
                pk = s16 | ((d16 - lo) << 14)
                plsc.store_compressed(cpak.at[pl.ds(cnt, L)], pk, mask=keep)
                cnt = cnt + plsc.all_reduce_population_count(keep)[0]
        return cnt

    cnt = lax.fori_loop(0, AR // 16, _blk, jnp.int32(0))

    def _prep(jj, par, pp, psem):
        for g in range(64 // L):
            off = jj * 64 + g * L
            pk = cpak[pl.ds(off, L)]
            s16 = pk & 16383
            d16 = lax.shift_right_logical(pk, 14)
            csrc2[par, pl.ds(g * L, L)] = s16
            cdst2[pp, pl.ds(par * 64 + g * L, L)] = d16
            sv = plsc.load_gather(st_tab, [s16])
            tv = plsc.load_gather(st_tab, [d16 + NP])
            e = sv + tv
            e = jnp.where(e >= 0, e, 0.01 * e)
            big = smax + tv
            mv = jnp.where(big >= 0, big, 0.01 * big)
            validf = jnp.where(off + ramp < cnt, 1.0, 0.0)
            wbuf[pl.ds(par * 64 + g * L, L)] = jnp.exp(e - mv) * validf
        pltpu.async_copy(z_hbm.at[csrc2.at[par]], rows_g.at[par], psem)

    def _wait(par, psem):
        pltpu.make_async_copy(z_hbm.at[csrc2.at[par]], rows_g.at[par],
                              psem).wait()

    def _scale(par):
        for gg in range(4):
            w16 = wbuf[pl.ds(par * 64 + gg * L, L)]
            for i in range(L):
                rloc = gg * L + i
                wr = w16[i]
                for q in range(D // L):
                    rows_s[par * 64 + rloc, pl.ds(q * L, L)] = (
                        rows_g[par, rloc, pl.ds(q * L, L)] * wr)
                rows_s[par * 64 + rloc, pl.ds(D, L)] = wr * onehot

    npairs = (cnt + 127) // 128

    def _wait_scatter(ppx):
        pltpu.make_async_copy(rows_s, acc.at[cdst2.at[ppx]], sem2).wait()

    @pl.when(npairs > 0)
    def _prologue():
        _prep(0, 0, jnp.int32(0), sem0)

    def _pair(p, carry):
        pp = p & 1
        _prep(2 * p + 1, 1, pp, sem1)
        _wait(0, sem0)

        @pl.when(p > 0)
        def _drain_prev():
            _wait_scatter(1 - pp)

        _scale(0)

        @pl.when(p + 1 < npairs)
        def _prefetch():
            _prep(2 * p + 2, 0, 1 - pp, sem0)

        _wait(1, sem1)
        _scale(1)
        pltpu.async_copy(rows_s, acc.at[cdst2.at[pp]], sem2, add=True)
        return carry

    lax.fori_loop(0, npairs, _pair, 0)

    @pl.when(npairs > 0)
    def _drain_last():
        _wait_scatter((npairs - 1) & 1)
    plsc.subcore_barrier()
    pltpu.sync_copy(acc.at[pl.ds(sid * tile_rows, tile_rows)],
                    part_hbm.at[pl.ds(c * HALF + sid * tile_rows, tile_rows)])


_edge_call = pl.kernel(
    _edge_body,
    out_type=jax.ShapeDtypeStruct((NP, WROW), jnp.float32),
    mesh=_mesh,
    compiler_params=_sc_params,
    scratch_types=[
        pltpu.VMEM((NP + HALF + L,), jnp.float32),
        pltpu.VMEM((16, 128), jnp.int32),
        pltpu.VMEM((16, 128), jnp.int32),
        pltpu.VMEM((CMAX,), jnp.int32),
        pltpu.VMEM((2, 64), jnp.int32),
        pltpu.VMEM((2, 128), jnp.int32),
        pltpu.VMEM((2, 64, D), jnp.float32),
        pltpu.VMEM((128, WROW), jnp.float32),
        pltpu.VMEM((128,), jnp.float32),
        pltpu.VMEM_SHARED((HALF, WROW), jnp.float32),
        pltpu.SemaphoreType.DMA,
        pltpu.SemaphoreType.DMA,
        pltpu.SemaphoreType.DMA,
    ],
)


def _pair_body(v_hbm, pair_hbm, out_hbm, v_tab, p0_buf, p1_buf, obuf):
    c = lax.axis_index("c")
    sid = lax.axis_index("s")
    wid = sid * NC + c
    base = wid * PPW

    pltpu.sync_copy(v_hbm, v_tab)
    pltpu.sync_copy(pair_hbm.at[0, pl.ds(base, PPW)], p0_buf)
    pltpu.sync_copy(pair_hbm.at[1, pl.ds(base, PPW)], p1_buf)

    ramp = lax.iota(jnp.int32, L)

    def _grp(k, carry):
        i0 = p0_buf[pl.ds(k * L, L)] * 4
        i1 = p1_buf[pl.ds(k * L, L)] * 4
        l0 = (plsc.load_gather(v_tab, [i0])
              + plsc.load_gather(v_tab, [i1 + 2]))
        l1 = (plsc.load_gather(v_tab, [i0 + 1])
              + plsc.load_gather(v_tab, [i1 + 3]))
        s0 = 1.0 / (1.0 + jnp.exp(-l0))
        s1 = 1.0 / (1.0 + jnp.exp(-l1))
        idx2 = (k * L + ramp) * 2
        plsc.store_scatter(obuf, [idx2], s0)
        plsc.store_scatter(obuf, [idx2 + 1], s1)
        return carry

    lax.fori_loop(0, PPW // L, _grp, 0)
    pltpu.sync_copy(obuf, out_hbm.at[pl.ds(base * 2, PPW * 2)])


_pair_call = pl.kernel(
    _pair_body,
    out_type=jax.ShapeDtypeStruct((2 * P,), jnp.float32),
    mesh=_mesh,
    compiler_params=_sc_params,
    scratch_types=[
        pltpu.VMEM((4 * NP,), jnp.float32),
        pltpu.VMEM((PPW,), jnp.int32),
        pltpu.VMEM((PPW,), jnp.int32),
        pltpu.VMEM((2 * PPW,), jnp.float32),
    ],
)


@jax.jit
def kernel(x, edge_index, pair_index, W1_0, A_0, W1_1, A_1, Wc, bc):
    xp = jnp.pad(x, ((0, NP - N), (0, 0)))
    pad_e = EPR * 128 - E
    src = jnp.pad(edge_index[0], (0, pad_e)).reshape(EPR, 128)
    dst = jnp.pad(edge_index[1], (0, pad_e)).reshape(EPR, 128)

    z1, stm1 = _dense_layer(xp, W1_0, A_0)
    part1 = _edge_call(stm1, src, dst, z1)
    h1 = _combine(part1)

    z2, stm2 = _dense_layer(h1, W1_1, A_1)
    part2 = _edge_call(stm2, src, dst, z2)
    h2 = _combine(part2)

    wstk = jnp.stack([Wc[0, :D], Wc[1, :D], Wc[0, D:], Wc[1, D:]], axis=1)
    bvec = jnp.concatenate([bc, jnp.zeros((2,), jnp.float32)])
    v = _classify(h2, wstk, bvec)

    probs_flat = _pair_call(v.reshape(-1), pair_index)
    return h2[:N], probs_flat.reshape(P, 2)


# speedup vs baseline: 2.2658x; 1.0173x over previous
import jax
import jax.numpy as jnp
from jax import lax
from jax.experimental import pallas as pl
from jax.experimental.pallas import tpu as pltpu
from jax.experimental.pallas import tpu_sc as plsc

N = 10000
NP = 10240
D = 128
E = 320000
NC, NS, L = 2, 16, 16
HALF = NP // NC
AR = 160
REAL_ROWS = E // 128
EPR = NS * AR
WROW = 144
CMAX = AR * 128 + 256
P = 65536
PPW = P // (NC * NS)

_mesh = plsc.VectorSubcoreMesh(
    core_axis_name="c", subcore_axis_name="s", num_cores=NC, num_subcores=NS)
_sc_params = pltpu.CompilerParams(
    needs_layout_passes=False, use_tc_tiling_on_sc=False)


def _dense_body(h_ref, w_ref, a_ref, z_ref, stm_ref):
    h = h_ref[...]
    w = w_ref[...]
    z = lax.dot_general(h, w, (((1,), (1,)), ((), ())),
                        preferred_element_type=jnp.float32)
    z_ref[...] = z
    a = a_ref[...]
    s = z @ a[0, :D]
    t = z @ a[0, D:]
    stm_ref[pl.ds(0, NP)] = s
    stm_ref[pl.ds(NP, NP)] = t
    stm_ref[pl.ds(2 * NP, L)] = jnp.broadcast_to(jnp.max(s), (L,))


def _dense_layer(h, w1, a):
    return pl.pallas_call(
        _dense_body,
        out_shape=[
            jax.ShapeDtypeStruct((NP, D), jnp.float32),
            jax.ShapeDtypeStruct((2 * NP + L,), jnp.float32),
        ],
    )(h, w1, a)


def _normalize(part):
    acc = part[...]
    num = acc[:, :D]
    den = acc[:, D:D + 1]
    h = jnp.where(den > 0, num / den, 0.0)
    return jnp.maximum(h, 0.0)


def _comb_dense_body(part_ref, w_ref, a_ref, z_ref, stm_ref):
    h = _normalize(part_ref)
    z = lax.dot_general(h, w_ref[...], (((1,), (1,)), ((), ())),
                        preferred_element_type=jnp.float32)
    z_ref[...] = z
    a = a_ref[...]
    s = z @ a[0, :D]
    t = z @ a[0, D:]
    stm_ref[pl.ds(0, NP)] = s
    stm_ref[pl.ds(NP, NP)] = t
    stm_ref[pl.ds(2 * NP, L)] = jnp.broadcast_to(jnp.max(s), (L,))


def _comb_dense(part, w1, a):
    return pl.pallas_call(
        _comb_dense_body,
        out_shape=[
            jax.ShapeDtypeStruct((NP, D), jnp.float32),
            jax.ShapeDtypeStruct((2 * NP + L,), jnp.float32),
        ],
    )(part, w1, a)


def _comb_classify_body(part_ref, wstk_ref, bvec_ref, h_ref, v_ref):
    h = _normalize(part_ref)
    h_ref[...] = h
    v = lax.dot_general(h, wstk_ref[...], (((1,), (0,)), ((), ())),
                        preferred_element_type=jnp.float32)
    v_ref[...] = v + bvec_ref[...][None, :]


def _comb_classify(part, wstk, bvec):
    return pl.pallas_call(
        _comb_classify_body,
        out_shape=[
            jax.ShapeDtypeStruct((NP, D), jnp.float32),
            jax.ShapeDtypeStruct((NP, 4), jnp.float32),
        ],
    )(part, wstk, bvec)


def _edge_body(stm_hbm, src_hbm, dst_hbm, z_hbm, part_hbm,
               st_tab, sstage, dstage, cpak, csrc2, cdst2,
               rows_g, rows_s, wbuf, acc, sem0, sem1, sem2):
    c = lax.axis_index("c")
    sid = lax.axis_index("s")
    lo = c * HALF

    pltpu.sync_copy(stm_hbm.at[pl.ds(0, NP)], st_tab.at[pl.ds(0, NP)])
    pltpu.sync_copy(stm_hbm.at[pl.ds(NP + lo, HALF)], st_tab.at[pl.ds(NP, HALF)])
    pltpu.sync_copy(stm_hbm.at[pl.ds(2 * NP, L)], st_tab.at[pl.ds(NP + HALF, L)])
    smax = st_tab[pl.ds(NP + HALF, L)][0]

    def _zero(r, carry):
        for q in range(WROW // L):
            rows_s[r, pl.ds(q * L, L)] = jnp.zeros((L,), jnp.float32)
        return carry
    lax.fori_loop(0, 128, _zero, 0)
    tile_rows = HALF // NS
    pltpu.sync_copy(rows_s, acc.at[pl.ds(sid * tile_rows, 128)])
    pltpu.sync_copy(rows_s, acc.at[pl.ds(sid * tile_rows + 128, 128)])
    pltpu.sync_copy(rows_s.at[pl.ds(0, 64)],
                    acc.at[pl.ds(sid * tile_rows + 256, 64)])
    plsc.subcore_barrier()

    ramp = lax.iota(jnp.int32, L)
    onehot = jnp.where(ramp == 0, 1.0, 0.0)

    zi = jnp.zeros((L,), jnp.int32)
    def _zc(b, carry):
        for q in range(8):
            cpak[pl.ds(b * 128 + q * L, L)] = zi
        return carry
    lax.fori_loop(0, CMAX // 128, _zc, 0)

    def _blk(blk, cnt):
        pltpu.sync_copy(src_hbm.at[pl.ds(sid * AR + blk * 16, 16)], sstage)
        pltpu.sync_copy(dst_hbm.at[pl.ds(sid * AR + blk * 16, 16)], dstage)

        for rr in range(16):
            grow = sid * AR + blk * 16 + rr
            growv = jnp.broadcast_to(grow, (L,))
            for g in range(128 // L):
                s16 = sstage[rr, pl.ds(g * L, L)]
                d16 = dstage[rr, pl.ds(g * L, L)]
                keep = ((d16 >= lo) & (d16 < lo + HALF)
                        & (growv < REAL_ROWS))
                pk = s16 | ((d16 - lo) << 14)
                plsc.store_compressed(cpak.at[pl.ds(cnt, L)], pk, mask=keep)
                cnt = cnt + plsc.all_reduce_population_count(keep)[0]
        return cnt

    cnt = lax.fori_loop(0, AR // 16, _blk, jnp.int32(0))

    def _prep(jj, par, pp, psem):
        for g in range(64 // L):
            off = jj * 64 + g * L
            pk = cpak[pl.ds(off, L)]
            s16 = pk & 16383
            d16 = lax.shift_right_logical(pk, 14)
            csrc2[par, pl.ds(g * L, L)] = s16
            cdst2[pp, pl.ds(par * 64 + g * L, L)] = d16
            sv = plsc.load_gather(st_tab, [s16])
            tv = plsc.load_gather(st_tab, [d16 + NP])
            e = sv + tv
            e = jnp.where(e >= 0, e, 0.01 * e)
            big = smax + tv
            mv = jnp.where(big >= 0, big, 0.01 * big)
            validf = jnp.where(off + ramp < cnt, 1.0, 0.0)
            wbuf[pl.ds(par * 64 + g * L, L)] = jnp.exp(e - mv) * validf
        pltpu.async_copy(z_hbm.at[csrc2.at[par]], rows_g.at[par], psem)

    def _wait(par, psem):
        pltpu.make_async_copy(z_hbm.at[csrc2.at[par]], rows_g.at[par],
                              psem).wait()

    def _scale(par):
        for gg in range(4):
            w16 = wbuf[pl.ds(par * 64 + gg * L, L)]
            for i in range(L):
                rloc = gg * L + i
                wr = w16[i]
                for q in range(D // L):
                    rows_s[par * 64 + rloc, pl.ds(q * L, L)] = (
                        rows_g[par, rloc, pl.ds(q * L, L)] * wr)
                rows_s[par * 64 + rloc, pl.ds(D, L)] = wr * onehot

    npairs = (cnt + 127) // 128

    def _wait_scatter(ppx):
        pltpu.make_async_copy(rows_s, acc.at[cdst2.at[ppx]], sem2).wait()

    @pl.when(npairs > 0)
    def _prologue():
        _prep(0, 0, jnp.int32(0), sem0)

    def _pair(p, carry):
        pp = p & 1
        _prep(2 * p + 1, 1, pp, sem1)
        _wait(0, sem0)

        @pl.when(p > 0)
        def _drain_prev():
            _wait_scatter(1 - pp)

        _scale(0)

        @pl.when(p + 1 < npairs)
        def _prefetch():
            _prep(2 * p + 2, 0, 1 - pp, sem0)

        _wait(1, sem1)
        _scale(1)
        pltpu.async_copy(rows_s, acc.at[cdst2.at[pp]], sem2, add=True)
        return carry

    lax.fori_loop(0, npairs, _pair, 0)

    @pl.when(npairs > 0)
    def _drain_last():
        _wait_scatter((npairs - 1) & 1)
    plsc.subcore_barrier()
    pltpu.sync_copy(acc.at[pl.ds(sid * tile_rows, tile_rows)],
                    part_hbm.at[pl.ds(c * HALF + sid * tile_rows, tile_rows)])


_edge_call = pl.kernel(
    _edge_body,
    out_type=jax.ShapeDtypeStruct((NP, WROW), jnp.float32),
    mesh=_mesh,
    compiler_params=_sc_params,
    scratch_types=[
        pltpu.VMEM((NP + HALF + L,), jnp.float32),
        pltpu.VMEM((16, 128), jnp.int32),
        pltpu.VMEM((16, 128), jnp.int32),
        pltpu.VMEM((CMAX,), jnp.int32),
        pltpu.VMEM((2, 64), jnp.int32),
        pltpu.VMEM((2, 128), jnp.int32),
        pltpu.VMEM((2, 64, D), jnp.float32),
        pltpu.VMEM((128, WROW), jnp.float32),
        pltpu.VMEM((128,), jnp.float32),
        pltpu.VMEM_SHARED((HALF, WROW), jnp.float32),
        pltpu.SemaphoreType.DMA,
        pltpu.SemaphoreType.DMA,
        pltpu.SemaphoreType.DMA,
    ],
)


def _pair_body(v_hbm, pair_hbm, out_hbm, v_tab, p0_buf, p1_buf, obuf):
    c = lax.axis_index("c")
    sid = lax.axis_index("s")
    wid = sid * NC + c
    base = wid * PPW

    pltpu.sync_copy(v_hbm, v_tab)
    pltpu.sync_copy(pair_hbm.at[0, pl.ds(base, PPW)], p0_buf)
    pltpu.sync_copy(pair_hbm.at[1, pl.ds(base, PPW)], p1_buf)

    ramp = lax.iota(jnp.int32, L)

    def _grp(k, carry):
        i0 = p0_buf[pl.ds(k * L, L)] * 4
        i1 = p1_buf[pl.ds(k * L, L)] * 4
        l0 = (plsc.load_gather(v_tab, [i0])
              + plsc.load_gather(v_tab, [i1 + 2]))
        l1 = (plsc.load_gather(v_tab, [i0 + 1])
              + plsc.load_gather(v_tab, [i1 + 3]))
        s0 = 1.0 / (1.0 + jnp.exp(-l0))
        s1 = 1.0 / (1.0 + jnp.exp(-l1))
        idx2 = (k * L + ramp) * 2
        plsc.store_scatter(obuf, [idx2], s0)
        plsc.store_scatter(obuf, [idx2 + 1], s1)
        return carry

    lax.fori_loop(0, PPW // L, _grp, 0)
    pltpu.sync_copy(obuf, out_hbm.at[pl.ds(base * 2, PPW * 2)])


_pair_call = pl.kernel(
    _pair_body,
    out_type=jax.ShapeDtypeStruct((2 * P,), jnp.float32),
    mesh=_mesh,
    compiler_params=_sc_params,
    scratch_types=[
        pltpu.VMEM((4 * NP,), jnp.float32),
        pltpu.VMEM((PPW,), jnp.int32),
        pltpu.VMEM((PPW,), jnp.int32),
        pltpu.VMEM((2 * PPW,), jnp.float32),
    ],
)


@jax.jit
def kernel(x, edge_index, pair_index, W1_0, A_0, W1_1, A_1, Wc, bc):
    xp = jnp.pad(x, ((0, NP - N), (0, 0)))
    pad_e = EPR * 128 - E
    src = jnp.pad(edge_index[0], (0, pad_e)).reshape(EPR, 128)
    dst = jnp.pad(edge_index[1], (0, pad_e)).reshape(EPR, 128)

    z1, stm1 = _dense_layer(xp, W1_0, A_0)
    part1 = _edge_call(stm1, src, dst, z1)

    z2, stm2 = _comb_dense(part1, W1_1, A_1)
    part2 = _edge_call(stm2, src, dst, z2)

    wstk = jnp.stack([Wc[0, :D], Wc[1, :D], Wc[0, D:], Wc[1, D:]], axis=1)
    bvec = jnp.concatenate([bc, jnp.zeros((2,), jnp.float32)])
    h2, v = _comb_classify(part2, wstk, bvec)

    probs_flat = _pair_call(v.reshape(-1), pair_index)
    return h2[:N], probs_flat.reshape(P, 2)
